# baseline (speedup 1.0000x reference)
import jax
import jax.numpy as jnp
from jax.experimental import pallas as pl

_DELTA = 0.05
_S = 128 * 128
_K = int(_S * 0.3 - 1) + 1
_HI0 = 0x3F800010


def _region_loss_kernel(t_ref, p_ref, out_ref):
    t = t_ref[...]
    p = p_ref[...]
    t0 = t[:, :_S]
    t1 = t[:, _S:2 * _S]
    t2 = t[:, 2 * _S:]
    gray = 0.39 * t0 + 0.5 * t1 + 0.11 * t2
    b = gray.shape[0]

    d = jnp.abs(t - p)
    c = jnp.minimum(d, _DELTA)
    f = 0.5 * c * c + _DELTA * (d - c)
    ones3s = jnp.ones((3 * _S, 8), jnp.float32)
    sf = jax.lax.dot_general(f, ones3s, (((1,), (0,)), ((), ())),
                             preferred_element_type=jnp.float32)[:, :1]
    fpix = f[:, :_S] + f[:, _S:2 * _S] + f[:, 2 * _S:]

    def _count(mid):
        vm = jax.lax.bitcast_convert_type(mid, jnp.float32)
        return jnp.sum((gray >= vm).astype(jnp.float32), axis=1, keepdims=True)

    def _probe(mid, lo, hi, cl, ch):
        cnt = _count(mid)
        ge = cnt >= _K
        lo = jnp.where(ge, mid, lo)
        cl = jnp.where(ge, cnt, cl)
        hi = jnp.where(ge, hi, mid)
        ch = jnp.where(ge, ch, cnt)
        return lo, hi, cl, ch

    def cond(carry):
        lo, hi, cl, ch, _ = carry
        return jnp.any(((hi - lo) > 1) & (cl != _K) & (ch != _K - 1))

    def body(carry):
        lo, hi, cl, ch, it = carry
        vlo = jax.lax.bitcast_convert_type(lo, jnp.float32)
        vhi = jax.lax.bitcast_convert_type(hi, jnp.float32)
        frac = (cl - _K) / jnp.maximum(cl - ch, 1.0)
        vm = vlo + (vhi - vlo) * frac
        m1 = jax.lax.bitcast_convert_type(vm, jnp.int32)
        m1 = jnp.clip(m1, lo + 1, hi - 1)
        lo, hi, cl, ch = _probe(m1, lo, hi, cl, ch)
        vlo = jax.lax.bitcast_convert_type(lo, jnp.float32)
        vhi = jax.lax.bitcast_convert_type(hi, jnp.float32)
        dens = jnp.maximum((cl - ch) / jnp.maximum(vhi - vlo, 1e-30), 1e-30)
        vm2 = vlo + (cl - _K) / dens
        mn2 = jnp.clip(jax.lax.bitcast_convert_type(vm2, jnp.int32),
                       lo + 1, hi - 1)
        mb2 = lo + ((hi - lo) >> 1)
        m2 = jnp.where((it & 1) == 0, mn2, mb2)
        lo, hi, cl, ch = _probe(m2, lo, hi, cl, ch)
        return lo, hi, cl, ch, it + 1

    lo0 = jnp.zeros((b, 1), jnp.int32)
    hi0 = jnp.full((b, 1), _HI0, jnp.int32)
    cl0 = jnp.full((b, 1), float(_S), jnp.float32)
    ch0 = jnp.zeros((b, 1), jnp.float32)
    lo, hi, cl, ch, _ = jax.lax.while_loop(
        cond, body, (lo0, hi0, cl0, ch0, jnp.int32(0)))
    vlo = jax.lax.bitcast_convert_type(lo, jnp.float32)
    vhi = jax.lax.bitcast_convert_type(hi, jnp.float32)
    mn = jnp.min(jnp.where(gray >= vlo, gray, 2.0), axis=1, keepdims=True)
    mx = jnp.max(jnp.where(gray < vhi, gray, -1.0), axis=1, keepdims=True)
    yu = jnp.where(cl == _K, mn, jnp.where(ch == _K - 1, mx, vlo))

    m = (gray >= yu).astype(jnp.float32)
    fm = fpix * m
    sfm = jax.lax.dot_general(fm, ones3s[:_S], (((1,), (0,)), ((), ())),
                              preferred_element_type=jnp.float32)[:, :1]
    out_ref[...] = jnp.sum(sf + 3.0 * sfm).reshape(1, 1, 1)


def kernel(pred, target):
    n, c, h, w = pred.shape
    s = h * w
    pr = pred.reshape(n, c * s)
    tr = target.reshape(n, c * s)
    blk = 16
    grid = n // blk
    partial = pl.pallas_call(
        _region_loss_kernel,
        grid=(grid,),
        in_specs=[
            pl.BlockSpec((blk, c * s), lambda i: (i, 0)),
            pl.BlockSpec((blk, c * s), lambda i: (i, 0)),
        ],
        out_specs=pl.BlockSpec((1, 1, 1), lambda i: (i, 0, 0)),
        out_shape=jax.ShapeDtypeStruct((grid, 1, 1), jnp.float32),
    )(tr, pr)
    return jnp.sum(partial) * (1.0 / (n * c * s))


# speedup vs baseline: 1.0143x; 1.0143x over previous
import jax
import jax.numpy as jnp
from jax.experimental import pallas as pl

_DELTA = 0.05
_S = 128 * 128
_K = int(_S * 0.3 - 1) + 1
_HI0 = 0x3F800010


def _region_loss_kernel(t_ref, p_ref, out_ref):
    t = t_ref[...]
    p = p_ref[...]
    t0 = t[:, :_S]
    t1 = t[:, _S:2 * _S]
    t2 = t[:, 2 * _S:]
    gray = 0.39 * t0 + 0.5 * t1 + 0.11 * t2
    b = gray.shape[0]

    d = jnp.abs(t - p)
    c = jnp.minimum(d, _DELTA)
    f = 0.5 * c * c + _DELTA * (d - c)
    fpix = f[:, :_S] + f[:, _S:2 * _S] + f[:, 2 * _S:]

    def _count(mid):
        vm = jax.lax.bitcast_convert_type(mid, jnp.float32)
        return jnp.sum((gray >= vm).astype(jnp.float32), axis=1, keepdims=True)

    def _probe(mid, lo, hi, cl, ch):
        cnt = _count(mid)
        ge = cnt >= _K
        lo = jnp.where(ge, mid, lo)
        cl = jnp.where(ge, cnt, cl)
        hi = jnp.where(ge, hi, mid)
        ch = jnp.where(ge, ch, cnt)
        return lo, hi, cl, ch

    def cond(carry):
        lo, hi, cl, ch, _ = carry
        return jnp.any(((hi - lo) > 1) & (cl != _K) & (ch != _K - 1))

    def body(carry):
        lo, hi, cl, ch, it = carry
        vlo = jax.lax.bitcast_convert_type(lo, jnp.float32)
        vhi = jax.lax.bitcast_convert_type(hi, jnp.float32)
        frac = (cl - _K) / jnp.maximum(cl - ch, 1.0)
        vm = vlo + (vhi - vlo) * frac
        m1 = jax.lax.bitcast_convert_type(vm, jnp.int32)
        m1 = jnp.clip(m1, lo + 1, hi - 1)
        lo, hi, cl, ch = _probe(m1, lo, hi, cl, ch)
        vlo = jax.lax.bitcast_convert_type(lo, jnp.float32)
        vhi = jax.lax.bitcast_convert_type(hi, jnp.float32)
        dens = jnp.maximum((cl - ch) / jnp.maximum(vhi - vlo, 1e-30), 1e-30)
        vm2 = vlo + (cl - _K) / dens
        mn2 = jnp.clip(jax.lax.bitcast_convert_type(vm2, jnp.int32),
                       lo + 1, hi - 1)
        mb2 = lo + ((hi - lo) >> 1)
        m2 = jnp.where((it & 1) == 0, mn2, mb2)
        lo, hi, cl, ch = _probe(m2, lo, hi, cl, ch)
        return lo, hi, cl, ch, it + 1

    lo0 = jnp.zeros((b, 1), jnp.int32)
    hi0 = jnp.full((b, 1), _HI0, jnp.int32)
    cl0 = jnp.full((b, 1), float(_S), jnp.float32)
    ch0 = jnp.zeros((b, 1), jnp.float32)
    lo, hi, cl, ch, _ = jax.lax.while_loop(
        cond, body, (lo0, hi0, cl0, ch0, jnp.int32(0)))
    vlo = jax.lax.bitcast_convert_type(lo, jnp.float32)
    vhi = jax.lax.bitcast_convert_type(hi, jnp.float32)
    mn = jnp.min(jnp.where(gray >= vlo, gray, 2.0), axis=1, keepdims=True)
    mx = jnp.max(jnp.where(gray < vhi, gray, -1.0), axis=1, keepdims=True)
    yu = jnp.where(cl == _K, mn, jnp.where(ch == _K - 1, mx, vlo))

    wp = jnp.where(gray >= yu, 4.0 * fpix, fpix)
    ones_s = jnp.ones((_S, 8), jnp.float32)
    sw = jax.lax.dot_general(wp, ones_s, (((1,), (0,)), ((), ())),
                             preferred_element_type=jnp.float32)[:, :1]
    out_ref[...] = jnp.sum(sw).reshape(1, 1, 1)


def kernel(pred, target):
    n, c, h, w = pred.shape
    s = h * w
    pr = pred.reshape(n, c * s)
    tr = target.reshape(n, c * s)
    blk = 32
    grid = n // blk
    partial = pl.pallas_call(
        _region_loss_kernel,
        grid=(grid,),
        in_specs=[
            pl.BlockSpec((blk, c * s), lambda i: (i, 0)),
            pl.BlockSpec((blk, c * s), lambda i: (i, 0)),
        ],
        out_specs=pl.BlockSpec((1, 1, 1), lambda i: (i, 0, 0)),
        out_shape=jax.ShapeDtypeStruct((grid, 1, 1), jnp.float32),
    )(tr, pr)
    return jnp.sum(partial) * (1.0 / (n * c * s))


# speedup vs baseline: 1.1550x; 1.1387x over previous
import jax
import jax.numpy as jnp
from jax.experimental import pallas as pl

_DELTA = 0.05
_S = 128 * 128
_K = int(_S * 0.3 - 1) + 1
_HI0 = 0x3F800010


def _region_loss_kernel(t_ref, p_ref, out_ref):
    t = t_ref[...]
    p = p_ref[...]
    t0 = t[:, :_S]
    t1 = t[:, _S:2 * _S]
    t2 = t[:, 2 * _S:]
    gray = 0.39 * t0 + 0.5 * t1 + 0.11 * t2
    b = gray.shape[0]

    d = jnp.abs(t - p)
    c = jnp.minimum(d, _DELTA)
    f = 0.5 * c * c + _DELTA * (d - c)
    ones3s = jnp.ones((3 * _S, 8), jnp.float32)
    sf = jax.lax.dot_general(f, ones3s, (((1,), (0,)), ((), ())),
                             preferred_element_type=jnp.float32)[:, :1]
    fpix = f[:, :_S] + f[:, _S:2 * _S] + f[:, 2 * _S:]

    def _count(mid):
        vm = jax.lax.bitcast_convert_type(mid, jnp.float32)
        return jnp.sum((gray >= vm).astype(jnp.float32), axis=1, keepdims=True)

    def _probe(mid, lo, hi, cl, ch):
        cnt = _count(mid)
        ge = cnt >= _K
        lo = jnp.where(ge, mid, lo)
        cl = jnp.where(ge, cnt, cl)
        hi = jnp.where(ge, hi, mid)
        ch = jnp.where(ge, ch, cnt)
        return lo, hi, cl, ch

    def cond(carry):
        lo, hi, cl, ch, _ = carry
        return jnp.any(((hi - lo) > 1) & (cl != _K) & (ch != _K - 1))

    def body(carry):
        lo, hi, cl, ch, it = carry
        vlo = jax.lax.bitcast_convert_type(lo, jnp.float32)
        vhi = jax.lax.bitcast_convert_type(hi, jnp.float32)
        frac = (cl - _K) / jnp.maximum(cl - ch, 1.0)
        vm = vlo + (vhi - vlo) * frac
        m1 = jax.lax.bitcast_convert_type(vm, jnp.int32)
        m1 = jnp.clip(m1, lo + 1, hi - 1)
        lo, hi, cl, ch = _probe(m1, lo, hi, cl, ch)
        vlo = jax.lax.bitcast_convert_type(lo, jnp.float32)
        vhi = jax.lax.bitcast_convert_type(hi, jnp.float32)
        dens = jnp.maximum((cl - ch) / jnp.maximum(vhi - vlo, 1e-30), 1e-30)
        vm2 = vlo + (cl - _K) / dens
        mn2 = jnp.clip(jax.lax.bitcast_convert_type(vm2, jnp.int32),
                       lo + 1, hi - 1)
        mb2 = lo + ((hi - lo) >> 1)
        m2 = jnp.where((it & 1) == 0, mn2, mb2)
        lo, hi, cl, ch = _probe(m2, lo, hi, cl, ch)
        return lo, hi, cl, ch, it + 1

    lo0 = jnp.zeros((b, 1), jnp.int32)
    hi0 = jnp.full((b, 1), _HI0, jnp.int32)
    cl0 = jnp.full((b, 1), float(_S), jnp.float32)
    ch0 = jnp.zeros((b, 1), jnp.float32)
    lo, hi, cl, ch, _ = jax.lax.while_loop(
        cond, body, (lo0, hi0, cl0, ch0, jnp.int32(0)))
    vlo = jax.lax.bitcast_convert_type(lo, jnp.float32)
    vhi = jax.lax.bitcast_convert_type(hi, jnp.float32)
    mn = jnp.min(jnp.where(gray >= vlo, gray, 2.0), axis=1, keepdims=True)
    mx = jnp.max(jnp.where(gray < vhi, gray, -1.0), axis=1, keepdims=True)
    yu = jnp.where(cl == _K, mn, jnp.where(ch == _K - 1, mx, vlo))

    m = (gray >= yu).astype(jnp.float32)
    fm = fpix * m
    sfm = jax.lax.dot_general(fm, ones3s[:_S], (((1,), (0,)), ((), ())),
                              preferred_element_type=jnp.float32)[:, :1]
    out_ref[...] = jnp.sum(sf + 3.0 * sfm).reshape(1, 1, 1)


def kernel(pred, target):
    n, c, h, w = pred.shape
    s = h * w
    pr = pred.reshape(n, c * s)
    tr = target.reshape(n, c * s)
    blk = 32
    grid = n // blk
    partial = pl.pallas_call(
        _region_loss_kernel,
        grid=(grid,),
        in_specs=[
            pl.BlockSpec((blk, c * s), lambda i: (i, 0)),
            pl.BlockSpec((blk, c * s), lambda i: (i, 0)),
        ],
        out_specs=pl.BlockSpec((1, 1, 1), lambda i: (i, 0, 0)),
        out_shape=jax.ShapeDtypeStruct((grid, 1, 1), jnp.float32),
    )(tr, pr)
    return jnp.sum(partial) * (1.0 / (n * c * s))
